# Initial kernel scaffold; baseline (speedup 1.0000x reference)
import jax
import jax.numpy as jnp
from jax.experimental import pallas as pl


def kernel(x, edge_index, W1, b1, W2, b2, Wc, bc):
    raise NotImplementedError("write your pallas kernel here")


# speedup vs baseline: 7.7960x; 7.7960x over previous
import functools

import jax
import jax.numpy as jnp
from jax import lax
from jax.experimental import pallas as pl
from jax.experimental.pallas import tpu as pltpu
from jax.experimental.pallas import tpu_sc as plsc

N = 10000
D = 128
E = 320000
NC = 2
NS = 16
NW = NC * NS
EPT = 10240
E_PAD = NW * EPT
C = 128
NCHUNK = EPT // C
RB = 624
TS = NS * RB
N_PAD = N + 8
R = 1000
GRID = N // R

_mesh = plsc.VectorSubcoreMesh(core_axis_name="c", subcore_axis_name="s")


HN = 5008
HRP = 640
ACC_R = 2 * HRP
WB = ACC_R // NS


@functools.partial(
    pl.kernel,
    out_type=jax.ShapeDtypeStruct((NC * ACC_R, D), jnp.float32),
    mesh=_mesh,
    compiler_params=pltpu.CompilerParams(needs_layout_passes=False),
    scratch_types=[
        pltpu.VMEM((EPT,), jnp.int32),
        pltpu.VMEM((HRP, D), jnp.float32),
        pltpu.VMEM((5, C), jnp.int32),
        pltpu.VMEM_SHARED((ACC_R, D), jnp.float32),
    ],
)
def _sc_deg(dst_hbm, iota_hbm, zeros_hbm, out_hbm, dstall, hist, idxb, acc):
    c = lax.axis_index("c")
    s = lax.axis_index("s")
    wid = c * NS + s
    ebase = wid * EPT
    rb = s * WB

    pltpu.sync_copy(zeros_hbm.at[pl.ds(0, WB)], hist.at[pl.ds(0, WB)])
    pltpu.sync_copy(hist.at[pl.ds(0, WB)], acc.at[pl.ds(rb, WB)])
    pltpu.sync_copy(dst_hbm.at[pl.ds(ebase, EPT)], dstall)
    plsc.subcore_barrier()

    lanes = lax.iota(jnp.int32, 16)
    onesv = jnp.ones((16,), jnp.float32)

    for r in (0, 1):
        lo = r * HN
        pltpu.sync_copy(zeros_hbm, hist)

        def sbody(i, carry):
            v = dstall[pl.ds(i * 16, 16)]
            m = (v >= lo) & (v < lo + HN)
            local = jnp.where(m, v - lo, 0)
            row = lax.shift_right_logical(local, 3)
            col = (local & 7) * 16 + lanes
            plsc.addupdate_scatter(hist, [row, col], onesv, mask=m)
            return carry

        lax.fori_loop(0, EPT // 16, sbody, 0)

        for k in range(5):
            pltpu.sync_copy(iota_hbm.at[pl.ds(r * HRP + k * C, C)], idxb.at[k])
            pltpu.sync_copy(hist.at[pl.ds(k * C, C)], acc.at[idxb.at[k]],
                            add=True)

    plsc.subcore_barrier()

    pltpu.sync_copy(acc.at[pl.ds(rb, WB)], hist.at[pl.ds(0, WB)])
    pltpu.sync_copy(hist.at[pl.ds(0, WB)], out_hbm.at[pl.ds(c * ACC_R + rb, WB)])


@functools.partial(
    pl.kernel,
    out_type=jax.ShapeDtypeStruct((NC * N, D), jnp.float32),
    mesh=_mesh,
    scratch_types=[
        pltpu.VMEM((2, C), jnp.int32),
        pltpu.VMEM((2, C), jnp.int32),
        pltpu.VMEM((2, C, D), jnp.float32),
        pltpu.VMEM_SHARED((N_PAD, D), jnp.float32),
        pltpu.SemaphoreType.DMA,
        pltpu.SemaphoreType.DMA,
    ],
)
def _sc_scatter(y_hbm, src_hbm, dst_hbm, zeros_hbm, out_hbm,
                srcb, dstb, rows, acc, sem0, sem1):
    c = lax.axis_index("c")
    s = lax.axis_index("s")
    wid = c * NS + s
    ebase = wid * EPT
    rbase = s * RB
    sems = (sem0, sem1)
    tail = RB - 4 * C

    pltpu.sync_copy(zeros_hbm, rows.at[0])
    for k in range(4):
        pltpu.sync_copy(rows.at[0], acc.at[pl.ds(rbase + k * C, C)])
    pltpu.sync_copy(rows.at[0, pl.ds(0, tail)],
                    acc.at[pl.ds(rbase + 4 * C, tail)])

    @pl.when(s == NS - 1)
    def _():
        pltpu.sync_copy(rows.at[0, pl.ds(0, N_PAD - TS)],
                        acc.at[pl.ds(TS, N_PAD - TS)])

    plsc.subcore_barrier()

    def body(g, carry):
        pltpu.sync_copy(src_hbm.at[pl.ds(ebase + g * C, C)], srcb.at[0])
        pltpu.sync_copy(dst_hbm.at[pl.ds(ebase + g * C, C)], dstb.at[0])
        pltpu.sync_copy(y_hbm.at[srcb.at[0]], rows.at[0])
        pltpu.sync_copy(rows.at[0], acc.at[dstb.at[0]], add=True)
        return carry

    lax.fori_loop(0, NCHUNK, body, 0)
    plsc.subcore_barrier()

    for k in range(4):
        pltpu.sync_copy(acc.at[pl.ds(rbase + k * C, C)], rows.at[0])
        pltpu.sync_copy(rows.at[0], out_hbm.at[pl.ds(c * N + rbase + k * C, C)])
    pltpu.sync_copy(acc.at[pl.ds(rbase + 4 * C, tail)], rows.at[0, pl.ds(0, tail)])
    pltpu.sync_copy(rows.at[0, pl.ds(0, tail)],
                    out_hbm.at[pl.ds(c * N + rbase + 4 * C, tail)])

    @pl.when(s == NS - 1)
    def _():
        pltpu.sync_copy(acc.at[pl.ds(TS, N - TS)], rows.at[0, pl.ds(0, N - TS)])
        pltpu.sync_copy(rows.at[0, pl.ds(0, N - TS)],
                        out_hbm.at[pl.ds(c * N + TS, N - TS)])


def _tc1_body(x_ref, w_ref, degp_ref, y_ref, dinv_ref):
    lanes = degp_ref[0] + degp_ref[1]
    deg = jnp.sum(lanes, axis=1, keepdims=True) + 1.0
    dinv = lax.rsqrt(deg)
    xw = jnp.dot(x_ref[...], w_ref[...], preferred_element_type=jnp.float32)
    y_ref[...] = xw * dinv
    dinv_ref[...] = dinv


def _tc1(x, W1, degp):
    return pl.pallas_call(
        _tc1_body,
        grid=(GRID,),
        in_specs=[
            pl.BlockSpec((R, D), lambda i: (i, 0)),
            pl.BlockSpec((D, D), lambda i: (0, 0)),
            pl.BlockSpec((NC, R, 16), lambda i: (0, i, 0)),
        ],
        out_specs=[
            pl.BlockSpec((R, D), lambda i: (i, 0)),
            pl.BlockSpec((R, 1), lambda i: (i, 0)),
        ],
        out_shape=[
            jax.ShapeDtypeStruct((N, D), jnp.float32),
            jax.ShapeDtypeStruct((N, 1), jnp.float32),
        ],
    )(x, W1, degp)


def _tc2_body(S_ref, y_ref, b_ref, w_ref, dinv_ref, out_ref):
    dinv = dinv_ref[...]
    h = jnp.tanh(dinv * (S_ref[0] + S_ref[1] + y_ref[...]) + b_ref[...])
    out_ref[...] = jnp.dot(h, w_ref[...],
                           preferred_element_type=jnp.float32) * dinv


def _tc2(S, y, b, W, dinv):
    return pl.pallas_call(
        _tc2_body,
        grid=(GRID,),
        in_specs=[
            pl.BlockSpec((NC, R, D), lambda i: (0, i, 0)),
            pl.BlockSpec((R, D), lambda i: (i, 0)),
            pl.BlockSpec((1, D), lambda i: (0, 0)),
            pl.BlockSpec((D, D), lambda i: (0, 0)),
            pl.BlockSpec((R, 1), lambda i: (i, 0)),
        ],
        out_specs=pl.BlockSpec((R, D), lambda i: (i, 0)),
        out_shape=jax.ShapeDtypeStruct((N, D), jnp.float32),
    )(S, y, b, W, dinv)


def _tc3_body(S_ref, y_ref, b_ref, w_ref, bc_ref, dinv_ref, out_ref):
    dinv = dinv_ref[...]
    h = jnp.tanh(dinv * (S_ref[0] + S_ref[1] + y_ref[...]) + b_ref[...])
    out_ref[...] = jnp.dot(h, w_ref[...],
                           preferred_element_type=jnp.float32) + bc_ref[...]


def _tc3(S, y, b, Wc, bc, dinv):
    return pl.pallas_call(
        _tc3_body,
        grid=(GRID,),
        in_specs=[
            pl.BlockSpec((NC, R, D), lambda i: (0, i, 0)),
            pl.BlockSpec((R, D), lambda i: (i, 0)),
            pl.BlockSpec((1, D), lambda i: (0, 0)),
            pl.BlockSpec((D, D), lambda i: (0, 0)),
            pl.BlockSpec((1, D), lambda i: (0, 0)),
            pl.BlockSpec((R, 1), lambda i: (i, 0)),
        ],
        out_specs=pl.BlockSpec((R, D), lambda i: (i, 0)),
        out_shape=jax.ShapeDtypeStruct((N, D), jnp.float32),
    )(S, y, b, Wc, bc, dinv)


@jax.jit
def kernel(x, edge_index, W1, b1, W2, b2, Wc, bc):
    src = edge_index[0].astype(jnp.int32)
    dst = edge_index[1].astype(jnp.int32)
    pad = E_PAD - E
    srcp = jnp.concatenate([src, jnp.zeros((pad,), jnp.int32)])
    dstp = jnp.concatenate([dst, jnp.full((pad,), N, jnp.int32)])
    iota = jnp.arange(ACC_R, dtype=jnp.int32)
    zerosD = jnp.zeros((C, D), jnp.float32)

    degw = _sc_deg(dstp, iota, jnp.zeros((HRP, D), jnp.float32))
    degp = degw.reshape(NC, 2, HRP, D)[:, :, : HN // 8, :]
    degp = degp.reshape(NC, 2 * HN, 16)[:, :N, :]
    y1, dinv = _tc1(x, W1, degp)
    S1 = _sc_scatter(y1, srcp, dstp, zerosD).reshape(NC, N, D)
    y2 = _tc2(S1, y1, b1.reshape(1, D), W2, dinv)
    S2 = _sc_scatter(y2, srcp, dstp, zerosD).reshape(NC, N, D)
    return _tc3(S2, y2, b2.reshape(1, D), Wc, bc.reshape(1, D), dinv)


# speedup vs baseline: 9.4272x; 1.2092x over previous
import functools

import jax
import jax.numpy as jnp
from jax import lax
from jax.experimental import pallas as pl
from jax.experimental.pallas import tpu as pltpu
from jax.experimental.pallas import tpu_sc as plsc

N = 10000
D = 128
E = 320000
NC = 2
NS = 16
NW = NC * NS
EPT = 10240
E_PAD = NW * EPT
C = 128
NCHUNK = EPT // C
RB = 624
TS = NS * RB
N_PAD = N + 8
R = 1000
GRID = N // R

_mesh = plsc.VectorSubcoreMesh(core_axis_name="c", subcore_axis_name="s")


HN = 5008
HRP = 640
ACC_R = 2 * HRP
WB = ACC_R // NS


@functools.partial(
    pl.kernel,
    out_type=jax.ShapeDtypeStruct((NC * ACC_R, D), jnp.float32),
    mesh=_mesh,
    compiler_params=pltpu.CompilerParams(needs_layout_passes=False),
    scratch_types=[
        pltpu.VMEM((EPT,), jnp.int32),
        pltpu.VMEM((HRP, D), jnp.float32),
        pltpu.VMEM((5, C), jnp.int32),
        pltpu.VMEM_SHARED((ACC_R, D), jnp.float32),
    ],
)
def _sc_deg(dst_hbm, iota_hbm, zeros_hbm, out_hbm, dstall, hist, idxb, acc):
    c = lax.axis_index("c")
    s = lax.axis_index("s")
    wid = c * NS + s
    ebase = wid * EPT
    rb = s * WB

    pltpu.sync_copy(zeros_hbm.at[pl.ds(0, WB)], hist.at[pl.ds(0, WB)])
    pltpu.sync_copy(hist.at[pl.ds(0, WB)], acc.at[pl.ds(rb, WB)])
    pltpu.sync_copy(dst_hbm.at[pl.ds(ebase, EPT)], dstall)
    plsc.subcore_barrier()

    lanes = lax.iota(jnp.int32, 16)
    onesv = jnp.ones((16,), jnp.float32)

    for r in (0, 1):
        lo = r * HN
        pltpu.sync_copy(zeros_hbm, hist)

        def sbody(i, carry):
            v = dstall[pl.ds(i * 16, 16)]
            m = (v >= lo) & (v < lo + HN)
            local = jnp.where(m, v - lo, 0)
            row = lax.shift_right_logical(local, 3)
            col = (local & 7) * 16 + lanes
            plsc.addupdate_scatter(hist, [row, col], onesv, mask=m)
            return carry

        lax.fori_loop(0, EPT // 16, sbody, 0)

        for k in range(5):
            pltpu.sync_copy(iota_hbm.at[pl.ds(r * HRP + k * C, C)], idxb.at[k])
            pltpu.sync_copy(hist.at[pl.ds(k * C, C)], acc.at[idxb.at[k]],
                            add=True)

    plsc.subcore_barrier()

    pltpu.sync_copy(acc.at[pl.ds(rb, WB)], hist.at[pl.ds(0, WB)])
    pltpu.sync_copy(hist.at[pl.ds(0, WB)], out_hbm.at[pl.ds(c * ACC_R + rb, WB)])


NBUF = 2
HC = 40


@functools.partial(
    pl.kernel,
    out_type=jax.ShapeDtypeStruct((NC * N, D), jnp.float32),
    mesh=_mesh,
    scratch_types=[
        pltpu.VMEM((HC, C), jnp.int32),
        pltpu.VMEM((HC, C), jnp.int32),
        pltpu.VMEM((NBUF, C, D), jnp.float32),
        pltpu.VMEM_SHARED((N_PAD, D), jnp.float32),
        pltpu.SemaphoreType.DMA,
        pltpu.SemaphoreType.DMA,
    ],
)
def _sc_scatter(y_hbm, src_hbm, dst_hbm, zeros_hbm, out_hbm,
                srcall, dstall, rows, acc, sem0, sem1):
    c = lax.axis_index("c")
    s = lax.axis_index("s")
    wid = c * NS + s
    rbase = s * RB
    sems = (sem0, sem1)
    tail = RB - 4 * C

    pltpu.sync_copy(zeros_hbm, rows.at[0])
    for k in range(4):
        pltpu.sync_copy(rows.at[0], acc.at[pl.ds(rbase + k * C, C)])
    pltpu.sync_copy(rows.at[0, pl.ds(0, tail)],
                    acc.at[pl.ds(rbase + 4 * C, tail)])

    @pl.when(s == NS - 1)
    def _():
        pltpu.sync_copy(rows.at[0, pl.ds(0, N_PAD - TS)],
                        acc.at[pl.ds(TS, N_PAD - TS)])

    plsc.subcore_barrier()

    def gather(b, g):
        return pltpu.make_async_copy(y_hbm.at[srcall.at[g]], rows.at[b],
                                     sems[b])

    for phase in range(NCHUNK // HC):
        pltpu.sync_copy(src_hbm.at[wid, pl.ds(phase * HC, HC)], srcall)
        pltpu.sync_copy(dst_hbm.at[wid, pl.ds(phase * HC, HC)], dstall)

        for b in range(NBUF):
            gather(b, b).start()

        def body(j, carry):
            for b in range(NBUF):
                g = NBUF * j + b
                gather(b, g).wait()
                pltpu.sync_copy(rows.at[b], acc.at[dstall.at[g]], add=True)
                gather(b, g + NBUF).start()
            return carry

        lax.fori_loop(0, HC // NBUF - 1, body, 0)
        for b in range(NBUF):
            g = HC - NBUF + b
            gather(b, g).wait()
            pltpu.sync_copy(rows.at[b], acc.at[dstall.at[g]], add=True)
    plsc.subcore_barrier()

    for k in range(4):
        pltpu.sync_copy(acc.at[pl.ds(rbase + k * C, C)], rows.at[0])
        pltpu.sync_copy(rows.at[0], out_hbm.at[pl.ds(c * N + rbase + k * C, C)])
    pltpu.sync_copy(acc.at[pl.ds(rbase + 4 * C, tail)], rows.at[0, pl.ds(0, tail)])
    pltpu.sync_copy(rows.at[0, pl.ds(0, tail)],
                    out_hbm.at[pl.ds(c * N + rbase + 4 * C, tail)])

    @pl.when(s == NS - 1)
    def _():
        pltpu.sync_copy(acc.at[pl.ds(TS, N - TS)], rows.at[0, pl.ds(0, N - TS)])
        pltpu.sync_copy(rows.at[0, pl.ds(0, N - TS)],
                        out_hbm.at[pl.ds(c * N + TS, N - TS)])


def _tc1_body(x_ref, w_ref, degp_ref, y_ref, dinv_ref):
    lanes = degp_ref[0] + degp_ref[1]
    deg = jnp.sum(lanes, axis=1, keepdims=True) + 1.0
    dinv = lax.rsqrt(deg)
    xw = jnp.dot(x_ref[...], w_ref[...], preferred_element_type=jnp.float32)
    y_ref[...] = xw * dinv
    dinv_ref[...] = dinv


def _tc1(x, W1, degp):
    return pl.pallas_call(
        _tc1_body,
        grid=(GRID,),
        in_specs=[
            pl.BlockSpec((R, D), lambda i: (i, 0)),
            pl.BlockSpec((D, D), lambda i: (0, 0)),
            pl.BlockSpec((NC, R, 16), lambda i: (0, i, 0)),
        ],
        out_specs=[
            pl.BlockSpec((R, D), lambda i: (i, 0)),
            pl.BlockSpec((R, 1), lambda i: (i, 0)),
        ],
        out_shape=[
            jax.ShapeDtypeStruct((N, D), jnp.float32),
            jax.ShapeDtypeStruct((N, 1), jnp.float32),
        ],
    )(x, W1, degp)


def _tc2_body(S_ref, y_ref, b_ref, w_ref, dinv_ref, out_ref):
    dinv = dinv_ref[...]
    h = jnp.tanh(dinv * (S_ref[0] + S_ref[1] + y_ref[...]) + b_ref[...])
    out_ref[...] = jnp.dot(h, w_ref[...],
                           preferred_element_type=jnp.float32) * dinv


def _tc2(S, y, b, W, dinv):
    return pl.pallas_call(
        _tc2_body,
        grid=(GRID,),
        in_specs=[
            pl.BlockSpec((NC, R, D), lambda i: (0, i, 0)),
            pl.BlockSpec((R, D), lambda i: (i, 0)),
            pl.BlockSpec((1, D), lambda i: (0, 0)),
            pl.BlockSpec((D, D), lambda i: (0, 0)),
            pl.BlockSpec((R, 1), lambda i: (i, 0)),
        ],
        out_specs=pl.BlockSpec((R, D), lambda i: (i, 0)),
        out_shape=jax.ShapeDtypeStruct((N, D), jnp.float32),
    )(S, y, b, W, dinv)


def _tc3_body(S_ref, y_ref, b_ref, w_ref, bc_ref, dinv_ref, out_ref):
    dinv = dinv_ref[...]
    h = jnp.tanh(dinv * (S_ref[0] + S_ref[1] + y_ref[...]) + b_ref[...])
    out_ref[...] = jnp.dot(h, w_ref[...],
                           preferred_element_type=jnp.float32) + bc_ref[...]


def _tc3(S, y, b, Wc, bc, dinv):
    return pl.pallas_call(
        _tc3_body,
        grid=(GRID,),
        in_specs=[
            pl.BlockSpec((NC, R, D), lambda i: (0, i, 0)),
            pl.BlockSpec((R, D), lambda i: (i, 0)),
            pl.BlockSpec((1, D), lambda i: (0, 0)),
            pl.BlockSpec((D, D), lambda i: (0, 0)),
            pl.BlockSpec((1, D), lambda i: (0, 0)),
            pl.BlockSpec((R, 1), lambda i: (i, 0)),
        ],
        out_specs=pl.BlockSpec((R, D), lambda i: (i, 0)),
        out_shape=jax.ShapeDtypeStruct((N, D), jnp.float32),
    )(S, y, b, Wc, bc, dinv)


@jax.jit
def kernel(x, edge_index, W1, b1, W2, b2, Wc, bc):
    src = edge_index[0].astype(jnp.int32)
    dst = edge_index[1].astype(jnp.int32)
    pad = E_PAD - E
    srcp = jnp.concatenate([src, jnp.zeros((pad,), jnp.int32)])
    dstp = jnp.concatenate([dst, jnp.full((pad,), N, jnp.int32)])
    iota = jnp.arange(ACC_R, dtype=jnp.int32)
    zerosD = jnp.zeros((C, D), jnp.float32)

    degw = _sc_deg(dstp, iota, jnp.zeros((HRP, D), jnp.float32))
    degp = degw.reshape(NC, 2, HRP, D)[:, :, : HN // 8, :]
    degp = degp.reshape(NC, 2 * HN, 16)[:, :N, :]
    y1, dinv = _tc1(x, W1, degp)
    src3 = srcp.reshape(NW, NCHUNK, C)
    dst3 = dstp.reshape(NW, NCHUNK, C)
    S1 = _sc_scatter(y1, src3, dst3, zerosD).reshape(NC, N, D)
    y2 = _tc2(S1, y1, b1.reshape(1, D), W2, dinv)
    S2 = _sc_scatter(y2, src3, dst3, zerosD).reshape(NC, N, D)
    return _tc3(S2, y2, b2.reshape(1, D), Wc, bc.reshape(1, D), dinv)


# speedup vs baseline: 27.9807x; 2.9681x over previous
import functools

import jax
import jax.numpy as jnp
from jax import lax
from jax.experimental import pallas as pl
from jax.experimental.pallas import tpu as pltpu
from jax.experimental.pallas import tpu_sc as plsc

N = 10000
D = 128
E = 320000
NC = 2
NS = 16
NW = NC * NS
EPT = 10240
E_PAD = NW * EPT
C = 128
NCHUNK = EPT // C
RB = 624
TS = NS * RB
N_PAD = N + 128
R = 1000
GRID = N // R

_mesh = plsc.VectorSubcoreMesh(core_axis_name="c", subcore_axis_name="s")


HN = 5008
HRP = 640
ACC_R = 2 * HRP
WB = ACC_R // NS


@functools.partial(
    pl.kernel,
    out_type=jax.ShapeDtypeStruct((NC * ACC_R, D), jnp.float32),
    mesh=_mesh,
    compiler_params=pltpu.CompilerParams(needs_layout_passes=False),
    scratch_types=[
        pltpu.VMEM((EPT,), jnp.int32),
        pltpu.VMEM((HRP, D), jnp.float32),
        pltpu.VMEM((5, C), jnp.int32),
        pltpu.VMEM_SHARED((ACC_R, D), jnp.float32),
    ],
)
def _sc_deg(dst_hbm, iota_hbm, zeros_hbm, out_hbm, dstall, hist, idxb, acc):
    c = lax.axis_index("c")
    s = lax.axis_index("s")
    wid = c * NS + s
    ebase = wid * EPT
    rb = s * WB

    pltpu.sync_copy(zeros_hbm.at[pl.ds(0, WB)], hist.at[pl.ds(0, WB)])
    pltpu.sync_copy(hist.at[pl.ds(0, WB)], acc.at[pl.ds(rb, WB)])
    pltpu.sync_copy(dst_hbm.at[pl.ds(ebase, EPT)], dstall)
    plsc.subcore_barrier()

    lanes = lax.iota(jnp.int32, 16)
    onesv = jnp.ones((16,), jnp.float32)

    for r in (0, 1):
        lo = r * HN
        pltpu.sync_copy(zeros_hbm, hist)

        def sbody(i, carry):
            v = dstall[pl.ds(i * 16, 16)]
            m = (v >= lo) & (v < lo + HN)
            local = jnp.where(m, v - lo, 0)
            row = lax.shift_right_logical(local, 3)
            col = (local & 7) * 16 + lanes
            plsc.addupdate_scatter(hist, [row, col], onesv, mask=m)
            return carry

        lax.fori_loop(0, EPT // 16, sbody, 0)

        for k in range(5):
            pltpu.sync_copy(iota_hbm.at[pl.ds(r * HRP + k * C, C)], idxb.at[k])
            pltpu.sync_copy(hist.at[pl.ds(k * C, C)], acc.at[idxb.at[k]],
                            add=True)

    plsc.subcore_barrier()

    pltpu.sync_copy(acc.at[pl.ds(rb, WB)], hist.at[pl.ds(0, WB)])
    pltpu.sync_copy(hist.at[pl.ds(0, WB)], out_hbm.at[pl.ds(c * ACC_R + rb, WB)])


NBUF = 2
HC = 40


@functools.partial(
    pl.kernel,
    out_type=jax.ShapeDtypeStruct((NC * N, D), jnp.float32),
    mesh=_mesh,
    scratch_types=[
        pltpu.VMEM((HC, C), jnp.int32),
        pltpu.VMEM((HC, C), jnp.int32),
        pltpu.VMEM((NBUF, C, D), jnp.float32),
        pltpu.VMEM_SHARED((N_PAD, D), jnp.float32),
        pltpu.SemaphoreType.DMA,
        pltpu.SemaphoreType.DMA,
    ],
)
def _sc_scatter(y_hbm, src_hbm, dst_hbm, zeros_hbm, out_hbm,
                srcall, dstall, rows, acc, sem0, sem1):
    c = lax.axis_index("c")
    s = lax.axis_index("s")
    wid = c * NS + s
    rbase = s * RB
    sems = (sem0, sem1)
    tail = RB - 4 * C

    pltpu.sync_copy(zeros_hbm, rows.at[0])
    for k in range(4):
        pltpu.sync_copy(rows.at[0], acc.at[pl.ds(rbase + k * C, C)])
    pltpu.sync_copy(rows.at[0, pl.ds(0, tail)],
                    acc.at[pl.ds(rbase + 4 * C, tail)])

    @pl.when(s == NS - 1)
    def _():
        pltpu.sync_copy(rows.at[0, pl.ds(0, N - TS)],
                        acc.at[pl.ds(TS, N - TS)])

    plsc.subcore_barrier()

    def gather(b, g):
        return pltpu.make_async_copy(y_hbm.at[srcall.at[g]], rows.at[b],
                                     sems[b])

    for phase in range(NCHUNK // HC):
        pltpu.sync_copy(src_hbm.at[wid, pl.ds(phase * HC, HC)], srcall)
        pltpu.sync_copy(dst_hbm.at[wid, pl.ds(phase * HC, HC)], dstall)

        for b in range(NBUF):
            gather(b, b).start()

        def body(j, carry):
            for b in range(NBUF):
                g = NBUF * j + b
                gather(b, g).wait()
                pltpu.sync_copy(rows.at[b], acc.at[dstall.at[g]], add=True)
                gather(b, g + NBUF).start()
            return carry

        lax.fori_loop(0, HC // NBUF - 1, body, 0)
        for b in range(NBUF):
            g = HC - NBUF + b
            gather(b, g).wait()
            pltpu.sync_copy(rows.at[b], acc.at[dstall.at[g]], add=True)
    plsc.subcore_barrier()

    for k in range(4):
        pltpu.sync_copy(acc.at[pl.ds(rbase + k * C, C)], rows.at[0])
        pltpu.sync_copy(rows.at[0], out_hbm.at[pl.ds(c * N + rbase + k * C, C)])
    pltpu.sync_copy(acc.at[pl.ds(rbase + 4 * C, tail)], rows.at[0, pl.ds(0, tail)])
    pltpu.sync_copy(rows.at[0, pl.ds(0, tail)],
                    out_hbm.at[pl.ds(c * N + rbase + 4 * C, tail)])

    @pl.when(s == NS - 1)
    def _():
        pltpu.sync_copy(acc.at[pl.ds(TS, N - TS)], rows.at[0, pl.ds(0, N - TS)])
        pltpu.sync_copy(rows.at[0, pl.ds(0, N - TS)],
                        out_hbm.at[pl.ds(c * N + TS, N - TS)])


def _tc1_body(x_ref, w_ref, degp_ref, y_ref, dinv_ref):
    lanes = degp_ref[0] + degp_ref[1]
    deg = jnp.sum(lanes, axis=1, keepdims=True) + 1.0
    dinv = lax.rsqrt(deg)
    xw = jnp.dot(x_ref[...], w_ref[...], preferred_element_type=jnp.float32)
    y_ref[...] = xw * dinv
    dinv_ref[...] = dinv


def _tc1(x, W1, degp):
    return pl.pallas_call(
        _tc1_body,
        grid=(GRID,),
        in_specs=[
            pl.BlockSpec((R, D), lambda i: (i, 0)),
            pl.BlockSpec((D, D), lambda i: (0, 0)),
            pl.BlockSpec((NC, R, 16), lambda i: (0, i, 0)),
        ],
        out_specs=[
            pl.BlockSpec((R, D), lambda i: (i, 0)),
            pl.BlockSpec((R, 1), lambda i: (i, 0)),
        ],
        out_shape=[
            jax.ShapeDtypeStruct((N, D), jnp.float32),
            jax.ShapeDtypeStruct((N, 1), jnp.float32),
        ],
    )(x, W1, degp)


def _tc2_body(S_ref, y_ref, b_ref, w_ref, dinv_ref, out_ref):
    dinv = dinv_ref[...]
    h = jnp.tanh(dinv * (S_ref[0] + S_ref[1] + y_ref[...]) + b_ref[...])
    out_ref[...] = jnp.dot(h, w_ref[...],
                           preferred_element_type=jnp.float32) * dinv


def _tc2(S, y, b, W, dinv):
    return pl.pallas_call(
        _tc2_body,
        grid=(GRID,),
        in_specs=[
            pl.BlockSpec((NC, R, D), lambda i: (0, i, 0)),
            pl.BlockSpec((R, D), lambda i: (i, 0)),
            pl.BlockSpec((1, D), lambda i: (0, 0)),
            pl.BlockSpec((D, D), lambda i: (0, 0)),
            pl.BlockSpec((R, 1), lambda i: (i, 0)),
        ],
        out_specs=pl.BlockSpec((R, D), lambda i: (i, 0)),
        out_shape=jax.ShapeDtypeStruct((N, D), jnp.float32),
    )(S, y, b, W, dinv)


def _tc3_body(S_ref, y_ref, b_ref, w_ref, bc_ref, dinv_ref, out_ref):
    dinv = dinv_ref[...]
    h = jnp.tanh(dinv * (S_ref[0] + S_ref[1] + y_ref[...]) + b_ref[...])
    out_ref[...] = jnp.dot(h, w_ref[...],
                           preferred_element_type=jnp.float32) + bc_ref[...]


def _tc3(S, y, b, Wc, bc, dinv):
    return pl.pallas_call(
        _tc3_body,
        grid=(GRID,),
        in_specs=[
            pl.BlockSpec((NC, R, D), lambda i: (0, i, 0)),
            pl.BlockSpec((R, D), lambda i: (i, 0)),
            pl.BlockSpec((1, D), lambda i: (0, 0)),
            pl.BlockSpec((D, D), lambda i: (0, 0)),
            pl.BlockSpec((1, D), lambda i: (0, 0)),
            pl.BlockSpec((R, 1), lambda i: (i, 0)),
        ],
        out_specs=pl.BlockSpec((R, D), lambda i: (i, 0)),
        out_shape=jax.ShapeDtypeStruct((N, D), jnp.float32),
    )(S, y, b, Wc, bc, dinv)


@jax.jit
def kernel(x, edge_index, W1, b1, W2, b2, Wc, bc):
    src = edge_index[0].astype(jnp.int32)
    dst = edge_index[1].astype(jnp.int32)
    pad = E_PAD - E
    padi = jnp.arange(pad, dtype=jnp.int32)
    srcp = jnp.concatenate([src, padi % N])
    dstp = jnp.concatenate([dst, N + padi % 128])
    iota = jnp.arange(ACC_R, dtype=jnp.int32)
    zerosD = jnp.zeros((C, D), jnp.float32)

    degw = _sc_deg(dstp, iota, jnp.zeros((HRP, D), jnp.float32))
    degp = degw.reshape(NC, 2, HRP, D)[:, :, : HN // 8, :]
    degp = degp.reshape(NC, 2 * HN, 16)[:, :N, :]
    y1, dinv = _tc1(x, W1, degp)
    src3 = srcp.reshape(NW, NCHUNK, C)
    dst3 = dstp.reshape(NW, NCHUNK, C)
    S1 = _sc_scatter(y1, src3, dst3, zerosD).reshape(NC, N, D)
    y2 = _tc2(S1, y1, b1.reshape(1, D), W2, dinv)
    S2 = _sc_scatter(y2, src3, dst3, zerosD).reshape(NC, N, D)
    return _tc3(S2, y2, b2.reshape(1, D), Wc, bc.reshape(1, D), dinv)
